# baseline (speedup 1.0000x reference)
import functools

import jax
import jax.numpy as jnp
from jax import lax
from jax.experimental import pallas as pl
from jax.experimental.pallas import tpu as pltpu
from jax.experimental.pallas import tpu_sc as plsc

_B, _S, _D = 2, 2048, 768
_E, _K = 64, 8
_H = 48
_DFF = _E * _H
_TOK_BLK = 1024
_N = _B * _S
_NW = 32
_TPW = _N // _NW


def _gate_t_body(x_ref, wg_ref, o_ref):
    o_ref[...] = jax.lax.dot_general(
        wg_ref[...], x_ref[...],
        dimension_numbers=(((1,), (1,)), ((), ())),
        preferred_element_type=jnp.float32)


def _tree(op, vals):
    vals = list(vals)
    while len(vals) > 1:
        nxt = [op(vals[i], vals[i + 1]) for i in range(0, len(vals) - 1, 2)]
        if len(vals) % 2:
            nxt.append(vals[-1])
        vals = nxt
    return vals[0]


def _sc_mask_kernel(gate_hbm, mask_hbm, gate_v, mask_v):
    wid = lax.axis_index("s") * 2 + lax.axis_index("c")
    base = wid * _TPW
    pltpu.sync_copy(gate_hbm.at[:, pl.ds(base, _TPW)], gate_v)

    neg = jnp.float32(-3.0e38)
    big = jnp.float32(_E)

    def body(g, carry):
        del carry
        sl = pl.ds(g * 16, 16)
        c = [gate_v[e, sl] for e in range(_E)]
        sel = [jnp.zeros((16,), jnp.float32) for _ in range(_E)]
        for _ in range(_K):
            m = _tree(jnp.maximum, c)
            idx = [jnp.where(c[e] == m, jnp.float32(e), big)
                   for e in range(_E)]
            imin = _tree(jnp.minimum, idx)
            for e in range(_E):
                hit = idx[e] == imin
                sel[e] = jnp.where(hit, jnp.float32(1.0), sel[e])
                c[e] = jnp.where(hit, neg, c[e])
        for e in range(_E):
            mask_v[e, sl] = sel[e]
        return 0

    lax.fori_loop(0, _TPW // 16, body, 0)
    pltpu.sync_copy(mask_v, mask_hbm.at[:, pl.ds(base, _TPW)])


def _ffn_body(x_ref, maskt_ref, w1_ref, w2_ref, exp_ref, o_ref):
    xb = x_ref[...]
    h = jnp.maximum(
        jax.lax.dot_general(xb, w1_ref[...],
                            dimension_numbers=(((1,), (1,)), ((), ())),
                            preferred_element_type=jnp.float32), 0.0)
    mexp = jax.lax.dot_general(
        maskt_ref[...], exp_ref[...],
        dimension_numbers=(((0,), (0,)), ((), ())),
        preferred_element_type=jnp.float32)
    o_ref[...] = jnp.dot(h * mexp, w2_ref[...],
                         preferred_element_type=jnp.float32)


@functools.partial(jax.jit, static_argnames=())
def kernel(x, wg, fc1_w, fc2_w):
    b, s, d = x.shape
    n = b * s
    xf = x.reshape(n, d)
    w1 = fc1_w.reshape(_DFF, d)
    w2 = fc2_w.transpose(0, 2, 1).reshape(_DFF, _D)
    expand = jnp.repeat(jnp.eye(_E, dtype=jnp.float32), _H, axis=1)

    gate_t = pl.pallas_call(
        _gate_t_body,
        grid=(n // _TOK_BLK,),
        in_specs=[
            pl.BlockSpec((_TOK_BLK, d), lambda i: (i, 0)),
            pl.BlockSpec((_E, d), lambda i: (0, 0)),
        ],
        out_specs=pl.BlockSpec((_E, _TOK_BLK), lambda i: (0, i)),
        out_shape=jax.ShapeDtypeStruct((_E, n), jnp.float32),
    )(xf, wg)

    mesh = plsc.VectorSubcoreMesh(core_axis_name="c", subcore_axis_name="s")
    mask_t = pl.kernel(
        _sc_mask_kernel,
        mesh=mesh,
        out_type=jax.ShapeDtypeStruct((_E, n), jnp.float32),
        scratch_types=[
            pltpu.VMEM((_E, _TPW), jnp.float32),
            pltpu.VMEM((_E, _TPW), jnp.float32),
        ],
    )(gate_t)

    yf = pl.pallas_call(
        _ffn_body,
        grid=(n // _TOK_BLK,),
        in_specs=[
            pl.BlockSpec((_TOK_BLK, d), lambda i: (i, 0)),
            pl.BlockSpec((_E, _TOK_BLK), lambda i: (0, i)),
            pl.BlockSpec((_DFF, d), lambda i: (0, 0)),
            pl.BlockSpec((_DFF, _D), lambda i: (0, 0)),
            pl.BlockSpec((_E, _DFF), lambda i: (0, 0)),
        ],
        out_specs=pl.BlockSpec((_TOK_BLK, _D), lambda i: (i, 0)),
        out_shape=jax.ShapeDtypeStruct((n, _D), jnp.float32),
    )(xf, mask_t, w1, w2, expand)
    return yf.reshape(b, s, _D)


# speedup vs baseline: 1.6313x; 1.6313x over previous
import functools

import jax
import jax.numpy as jnp
from jax.experimental import pallas as pl
from jax.experimental.pallas import tpu as pltpu

_B, _S, _D = 2, 2048, 768
_E, _K = 64, 8
_H = 48
_DFF = _E * _H
_TOK_BLK = 1024


def _ffn_body(x_ref, wgt_ref, w1_ref, w2_ref, exp_ref, o_ref):
    xb = x_ref[...]
    g = jnp.dot(xb, wgt_ref[...], preferred_element_type=jnp.float32)
    h = jnp.maximum(
        jax.lax.dot_general(xb, w1_ref[...],
                            dimension_numbers=(((1,), (1,)), ((), ())),
                            preferred_element_type=jnp.float32), 0.0)
    iota = jax.lax.broadcasted_iota(jnp.int32, g.shape, 1).astype(jnp.float32)
    neg = jnp.float32(jnp.finfo(jnp.float32).min)
    gcur = g
    sel = jnp.zeros(g.shape, dtype=jnp.bool_)
    for _ in range(_K):
        m = jnp.max(gcur, axis=1, keepdims=True)
        eq = gcur == m
        jfirst = jnp.min(jnp.where(eq, iota, jnp.float32(_E)), axis=1,
                         keepdims=True)
        first = iota == jfirst
        sel = sel | first
        gcur = jnp.where(first, neg, gcur)
    mask = sel.astype(jnp.bfloat16)
    mexp = jnp.dot(mask, exp_ref[...], preferred_element_type=jnp.float32)
    o_ref[...] = jnp.dot(h * mexp, w2_ref[...],
                         preferred_element_type=jnp.float32)


@functools.partial(jax.jit, static_argnames=())
def kernel(x, wg, fc1_w, fc2_w):
    b, s, d = x.shape
    n = b * s
    xf = x.reshape(n, d)
    wgt = wg.T
    w1 = fc1_w.reshape(_DFF, d)
    w2 = fc2_w.transpose(0, 2, 1).reshape(_DFF, _D)
    expand = jnp.repeat(jnp.eye(_E, dtype=jnp.bfloat16), _H, axis=1)

    yf = pl.pallas_call(
        _ffn_body,
        grid=(n // _TOK_BLK,),
        in_specs=[
            pl.BlockSpec((_TOK_BLK, d), lambda i: (i, 0)),
            pl.BlockSpec((d, _E), lambda i: (0, 0)),
            pl.BlockSpec((_DFF, d), lambda i: (0, 0)),
            pl.BlockSpec((_DFF, _D), lambda i: (0, 0)),
            pl.BlockSpec((_E, _DFF), lambda i: (0, 0)),
        ],
        out_specs=pl.BlockSpec((_TOK_BLK, _D), lambda i: (i, 0)),
        out_shape=jax.ShapeDtypeStruct((n, _D), jnp.float32),
    )(xf, wgt, w1, w2, expand)
    return yf.reshape(b, s, _D)


# speedup vs baseline: 1.6813x; 1.0306x over previous
import functools

import jax
import jax.numpy as jnp
from jax.experimental import pallas as pl

_B, _S, _D = 2, 2048, 768
_E, _K = 64, 8
_H = 48
_DFF = _E * _H
_TOK_BLK = 1024


def _ffn_body(x_ref, wgt_ref, w1_ref, w2_ref, exp_ref, o_ref):
    xb = x_ref[...]
    g = jax.lax.dot_general(xb, wgt_ref[...],
                            dimension_numbers=(((1,), (1,)), ((), ())),
                            preferred_element_type=jnp.float32)
    h = jnp.maximum(
        jax.lax.dot_general(xb, w1_ref[...],
                            dimension_numbers=(((1,), (1,)), ((), ())),
                            preferred_element_type=jnp.float32), 0.0)
    iota = jax.lax.broadcasted_iota(jnp.int32, g.shape, 1).astype(jnp.float32)
    neg = jnp.float32(jnp.finfo(jnp.float32).min)
    gcur = g
    sel = jnp.zeros(g.shape, dtype=jnp.bool_)
    for _ in range(_K):
        m = jnp.max(gcur, axis=1, keepdims=True)
        eq = gcur == m
        jfirst = jnp.min(jnp.where(eq, iota, jnp.float32(_E)), axis=1,
                         keepdims=True)
        first = iota == jfirst
        sel = sel | first
        gcur = jnp.where(first, neg, gcur)
    mask = sel.astype(jnp.bfloat16)
    mexp = jnp.dot(mask, exp_ref[...], preferred_element_type=jnp.float32)
    o_ref[...] = jnp.dot(h * mexp, w2_ref[...],
                         preferred_element_type=jnp.float32)


@functools.partial(jax.jit, static_argnames=())
def kernel(x, wg, fc1_w, fc2_w):
    b, s, d = x.shape
    n = b * s
    xf = x.reshape(n, d)
    w1 = fc1_w.reshape(_DFF, d)
    w2 = fc2_w.transpose(0, 2, 1).reshape(_DFF, _D)
    expand = jnp.repeat(jnp.eye(_E, dtype=jnp.bfloat16), _H, axis=1)

    yf = pl.pallas_call(
        _ffn_body,
        grid=(n // _TOK_BLK,),
        in_specs=[
            pl.BlockSpec((_TOK_BLK, d), lambda i: (i, 0)),
            pl.BlockSpec((_E, d), lambda i: (0, 0)),
            pl.BlockSpec((_DFF, d), lambda i: (0, 0)),
            pl.BlockSpec((_DFF, _D), lambda i: (0, 0)),
            pl.BlockSpec((_E, _DFF), lambda i: (0, 0)),
        ],
        out_specs=pl.BlockSpec((_TOK_BLK, _D), lambda i: (i, 0)),
        out_shape=jax.ShapeDtypeStruct((n, _D), jnp.float32),
    )(xf, wg, w1, w2, expand)
    return yf.reshape(b, s, _D)
